# baseline (device time: 15418 ns/iter reference)
import jax
import jax.numpy as jnp
from jax import lax
from jax.experimental import pallas as pl
from jax.experimental.pallas import tpu as pltpu

T = 256
D = 512
V_LOCAL = 4096
N_CHUNKS = 8
C = V_LOCAL // N_CHUNKS


def kernel(x, W, labels):
    def body(x_ref, w_ref, lab_ref, out_ref, acc_ref, send_ref, recv_ref,
             send_sem, recv_sem):
        i = pl.program_id(0)
        my_x = lax.axis_index("x")
        my_y = lax.axis_index("y")
        peer = (1 - my_x, my_y)
        barrier_sem = pltpu.get_barrier_semaphore()

        @pl.when(i == 0)
        def _():
            pl.semaphore_signal(barrier_sem, inc=1, device_id=peer,
                                device_id_type=pl.DeviceIdType.MESH)
            acc_ref[0, :] = jnp.zeros((T,), jnp.float32)
            acc_ref[1, :] = jnp.zeros((T,), jnp.float32)

        logits = jnp.dot(x_ref[:, :], w_ref[:, :],
                         preferred_element_type=jnp.float32)
        s = jnp.sum(jnp.exp(logits), axis=1)
        lab_local = lab_ref[:] - my_x * V_LOCAL - i * C
        col = lax.broadcasted_iota(jnp.int32, (T, C), 1)
        g = jnp.sum(jnp.where(col == lab_local[:, None], logits, 0.0),
                    axis=1)
        acc_ref[0, :] += s
        acc_ref[1, :] += g

        @pl.when(i == N_CHUNKS - 1)
        def _():
            send_ref[:, :] = acc_ref[:, :]
            pl.semaphore_wait(barrier_sem, 1)
            rdma = pltpu.make_async_remote_copy(
                src_ref=send_ref,
                dst_ref=recv_ref,
                send_sem=send_sem,
                recv_sem=recv_sem,
                device_id=peer,
                device_id_type=pl.DeviceIdType.MESH,
            )
            rdma.start()
            rdma.wait()
            out_ref[:] = (jnp.log(acc_ref[0, :] + recv_ref[0, :])
                          - (acc_ref[1, :] + recv_ref[1, :]))

    return pl.pallas_call(
        body,
        grid=(N_CHUNKS,),
        out_shape=jax.ShapeDtypeStruct((T,), jnp.float32),
        in_specs=[
            pl.BlockSpec((T, D), lambda i: (0, 0)),
            pl.BlockSpec((D, C), lambda i: (0, i)),
            pl.BlockSpec((T,), lambda i: (0,)),
        ],
        out_specs=pl.BlockSpec((T,), lambda i: (0,)),
        scratch_shapes=[
            pltpu.VMEM((2, T), jnp.float32),
            pltpu.VMEM((2, T), jnp.float32),
            pltpu.VMEM((2, T), jnp.float32),
            pltpu.SemaphoreType.DMA,
            pltpu.SemaphoreType.DMA,
        ],
        compiler_params=pltpu.CompilerParams(
            collective_id=0,
            dimension_semantics=("arbitrary",),
        ),
    )(x, W, labels)


# device time: 12247 ns/iter; 1.2589x vs baseline; 1.2589x over previous
import jax
import jax.numpy as jnp
from jax import lax
from jax.experimental import pallas as pl
from jax.experimental.pallas import tpu as pltpu

T = 256
D = 512
V_LOCAL = 4096


def kernel(x, W, labels):
    def body(x_hbm, w_hbm, lab_ref, out_ref, x_vmem, w_vmem,
             send_ref, recv_ref, cx_sem, cw_sem, send_sem, recv_sem):
        my_x = lax.axis_index("x")
        my_y = lax.axis_index("y")
        peer = (1 - my_x, my_y)

        barrier_sem = pltpu.get_barrier_semaphore()
        pl.semaphore_signal(barrier_sem, inc=1, device_id=peer,
                            device_id_type=pl.DeviceIdType.MESH)

        cx = pltpu.make_async_copy(x_hbm, x_vmem, cx_sem)
        cw = pltpu.make_async_copy(w_hbm, w_vmem, cw_sem)
        cx.start()
        cw.start()
        cx.wait()
        cw.wait()

        logits = jnp.dot(x_vmem[:, :], w_vmem[:, :],
                         preferred_element_type=jnp.float32)
        s = jnp.sum(jnp.exp(logits), axis=1)
        lab_local = lab_ref[:] - my_x * V_LOCAL
        col = lax.broadcasted_iota(jnp.int32, (T, V_LOCAL), 1)
        g = jnp.sum(jnp.where(col == lab_local[:, None], logits, 0.0),
                    axis=1)

        send_ref[0, :] = s
        send_ref[1, :] = g

        pl.semaphore_wait(barrier_sem, 1)
        rdma = pltpu.make_async_remote_copy(
            src_ref=send_ref,
            dst_ref=recv_ref,
            send_sem=send_sem,
            recv_sem=recv_sem,
            device_id=peer,
            device_id_type=pl.DeviceIdType.MESH,
        )
        rdma.start()
        rdma.wait_recv()
        out_ref[:] = (jnp.log(s + recv_ref[0, :])
                      - (g + recv_ref[1, :]))
        rdma.wait_send()

    return pl.pallas_call(
        body,
        out_shape=jax.ShapeDtypeStruct((T,), jnp.float32),
        in_specs=[
            pl.BlockSpec(memory_space=pltpu.MemorySpace.HBM),
            pl.BlockSpec(memory_space=pltpu.MemorySpace.HBM),
            pl.BlockSpec(memory_space=pltpu.VMEM),
        ],
        out_specs=pl.BlockSpec(memory_space=pltpu.VMEM),
        scratch_shapes=[
            pltpu.VMEM((T, D), jnp.float32),
            pltpu.VMEM((D, V_LOCAL), jnp.float32),
            pltpu.VMEM((2, T), jnp.float32),
            pltpu.VMEM((2, T), jnp.float32),
            pltpu.SemaphoreType.DMA,
            pltpu.SemaphoreType.DMA,
            pltpu.SemaphoreType.DMA,
            pltpu.SemaphoreType.DMA,
        ],
        compiler_params=pltpu.CompilerParams(collective_id=0),
    )(x, W, labels)


# device time: 11867 ns/iter; 1.2992x vs baseline; 1.0320x over previous
import jax
import jax.numpy as jnp
from jax import lax
from jax.experimental import pallas as pl
from jax.experimental.pallas import tpu as pltpu

T = 256
H = T // 2
V_LOCAL = 4096


def kernel(x, W, labels):
    def body(x_ref, w_ref, lab_ref, out_ref, send0, send1, recv0, recv1,
             send_sems, recv_sems):
        my_x = lax.axis_index("x")
        my_y = lax.axis_index("y")
        peer = (1 - my_x, my_y)

        barrier_sem = pltpu.get_barrier_semaphore()
        pl.semaphore_signal(barrier_sem, inc=1, device_id=peer,
                            device_id_type=pl.DeviceIdType.MESH)

        logits = jnp.dot(x_ref[:, :], w_ref[:, :],
                         preferred_element_type=jnp.float32)
        lab_local = lab_ref[:] - my_x * V_LOCAL
        col = lax.broadcasted_iota(jnp.int32, (H, V_LOCAL), 1)

        l0 = logits[:H, :]
        s0 = jnp.sum(jnp.exp(l0), axis=1)
        g0 = jnp.sum(jnp.where(col == lab_local[:H, None], l0, 0.0), axis=1)
        send0[0, :] = s0
        send0[1, :] = g0
        pl.semaphore_wait(barrier_sem, 1)
        rdma0 = pltpu.make_async_remote_copy(
            src_ref=send0, dst_ref=recv0,
            send_sem=send_sems.at[0], recv_sem=recv_sems.at[0],
            device_id=peer, device_id_type=pl.DeviceIdType.MESH,
        )
        rdma0.start()

        l1 = logits[H:, :]
        s1 = jnp.sum(jnp.exp(l1), axis=1)
        g1 = jnp.sum(jnp.where(col == lab_local[H:, None], l1, 0.0), axis=1)
        send1[0, :] = s1
        send1[1, :] = g1
        rdma1 = pltpu.make_async_remote_copy(
            src_ref=send1, dst_ref=recv1,
            send_sem=send_sems.at[1], recv_sem=recv_sems.at[1],
            device_id=peer, device_id_type=pl.DeviceIdType.MESH,
        )
        rdma1.start()

        rdma0.wait_recv()
        out_ref[:H] = jnp.log(s0 + recv0[0, :]) - (g0 + recv0[1, :])
        rdma1.wait_recv()
        out_ref[H:] = jnp.log(s1 + recv1[0, :]) - (g1 + recv1[1, :])
        rdma0.wait_send()
        rdma1.wait_send()

    return pl.pallas_call(
        body,
        out_shape=jax.ShapeDtypeStruct((T,), jnp.float32),
        in_specs=[
            pl.BlockSpec(memory_space=pltpu.VMEM),
            pl.BlockSpec(memory_space=pltpu.VMEM),
            pl.BlockSpec(memory_space=pltpu.VMEM),
        ],
        out_specs=pl.BlockSpec(memory_space=pltpu.VMEM),
        scratch_shapes=[
            pltpu.VMEM((2, H), jnp.float32),
            pltpu.VMEM((2, H), jnp.float32),
            pltpu.VMEM((2, H), jnp.float32),
            pltpu.VMEM((2, H), jnp.float32),
            pltpu.SemaphoreType.DMA((2,)),
            pltpu.SemaphoreType.DMA((2,)),
        ],
        compiler_params=pltpu.CompilerParams(collective_id=0),
    )(x, W, labels)


# device time: 11603 ns/iter; 1.3288x vs baseline; 1.0228x over previous
import jax
import jax.numpy as jnp
from jax import lax
from jax.experimental import pallas as pl
from jax.experimental.pallas import tpu as pltpu

T = 256
V_LOCAL = 4096


def kernel(x, W, labels):
    def body(x_ref, w_ref, lab_ref, out_ref, send_ref, recv_ref,
             send_sem, recv_sem):
        my_x = lax.axis_index("x")
        my_y = lax.axis_index("y")
        peer = (1 - my_x, my_y)

        barrier_sem = pltpu.get_barrier_semaphore()
        pl.semaphore_signal(barrier_sem, inc=1, device_id=peer,
                            device_id_type=pl.DeviceIdType.MESH)

        logits = jnp.dot(x_ref[:, :], w_ref[:, :],
                         preferred_element_type=jnp.float32)
        s = jnp.sum(jnp.exp(logits), axis=1)
        lab_local = lab_ref[:] - my_x * V_LOCAL
        col = lax.broadcasted_iota(jnp.int32, (T, V_LOCAL), 1)
        g = jnp.sum(jnp.where(col == lab_local[:, None], logits, 0.0),
                    axis=1)

        send_ref[0, :] = s
        send_ref[1, :] = g

        pl.semaphore_wait(barrier_sem, 1)
        rdma = pltpu.make_async_remote_copy(
            src_ref=send_ref,
            dst_ref=recv_ref,
            send_sem=send_sem,
            recv_sem=recv_sem,
            device_id=peer,
            device_id_type=pl.DeviceIdType.MESH,
        )
        rdma.start()
        rdma.wait_recv()
        out_ref[:] = (jnp.log(s + recv_ref[0, :])
                      - (g + recv_ref[1, :]))
        rdma.wait_send()

    return pl.pallas_call(
        body,
        out_shape=jax.ShapeDtypeStruct((T,), jnp.float32),
        in_specs=[
            pl.BlockSpec(memory_space=pltpu.VMEM),
            pl.BlockSpec(memory_space=pltpu.VMEM),
            pl.BlockSpec(memory_space=pltpu.VMEM),
        ],
        out_specs=pl.BlockSpec(memory_space=pltpu.VMEM),
        scratch_shapes=[
            pltpu.VMEM((2, T), jnp.float32),
            pltpu.VMEM((2, T), jnp.float32),
            pltpu.SemaphoreType.DMA,
            pltpu.SemaphoreType.DMA,
        ],
        compiler_params=pltpu.CompilerParams(collective_id=0),
    )(x, W, labels)
